# baseline (speedup 1.0000x reference)
import functools

import jax
import jax.numpy as jnp
from jax.experimental import pallas as pl
from jax.experimental.pallas import tpu as pltpu

B, S, D, K, V, GMAX = 16, 2048, 128, 4, 256, 512
NL = K - 1
SP = S + 8


def _gbst_kernel(idxT_ref, emb_ref, wk_ref, projT_ref,
                 swT_ref, sb_ref, out_ref, esp_ref):
    f32 = jnp.float32

    seq_col = idxT_ref[0, :, 0:1]
    vio = jax.lax.broadcasted_iota(jnp.int32, (S, V), 1)
    ohe = (seq_col == vio).astype(f32)
    esp_ref[pl.ds(0, S), :] = jnp.dot(
        ohe, emb_ref[...], preferred_element_type=f32)
    esp_ref[pl.ds(S, SP - S), :] = jnp.zeros((SP - S, D), f32)

    conv = esp_ref[pl.ds(0, S), :] * wk_ref[0:1, :]
    for k in range(1, K):
        conv = conv + esp_ref[pl.ds(k, S), :] * wk_ref[k:k + 1, :]
    es2 = jnp.dot(conv, projT_ref[...], preferred_element_type=f32)

    gio = jax.lax.broadcasted_iota(jnp.int32, (S, GMAX), 1)
    ohT = jnp.concatenate(
        [(idxT_ref[0, :, l + 1:l + 2] == gio + 1).astype(f32)
         for l in range(NL)], axis=1)
    dn = (((0,), (0,)), ((), ()))
    sums = jax.lax.dot_general(ohT, es2, dn,
                               preferred_element_type=f32)
    cnt_col = jax.lax.dot_general(ohT, jnp.ones((S, 1), f32), dn,
                                  preferred_element_type=f32)
    mean = sums * (1.0 / jnp.maximum(cnt_col, 1.0))
    mscore = jnp.dot(mean, swT_ref[...],
                     preferred_element_type=f32)

    cnt_rows = jnp.concatenate(
        [jnp.sum(ohT[:, l * GMAX:(l + 1) * GMAX], axis=0, keepdims=True)
         for l in range(NL)], axis=0)
    gi = jax.lax.broadcasted_iota(jnp.int32, (GMAX, GMAX), 0)
    gj = jax.lax.broadcasted_iota(jnp.int32, (GMAX, GMAX), 1)
    tri = (gi <= gj).astype(f32)
    cum_rows = jnp.dot(cnt_rows, tri, preferred_element_type=f32)

    sio = jax.lax.broadcasted_iota(jnp.int32, (S, GMAX), 0)
    s_pos = sio.astype(f32)
    oh2 = jnp.concatenate(
        [((s_pos >= cum_rows[l:l + 1] - cnt_rows[l:l + 1])
          & (s_pos < cum_rows[l:l + 1])).astype(f32)
         for l in range(NL)], axis=1)

    neg = -jnp.finfo(f32).max
    bias = sb_ref[0, 0]
    sc0 = jnp.dot(es2, swT_ref[...], preferred_element_type=f32) + bias
    mask0 = idxT_ref[0, :, 0:1] == 0
    scores = [jnp.where(mask0, neg, sc0)]
    for l in range(NL):
        scl = jnp.dot(oh2[:, l * GMAX:(l + 1) * GMAX],
                      mscore[l * GMAX:(l + 1) * GMAX, :],
                      preferred_element_type=f32) + bias
        maskl = idxT_ref[0, :, l + 1:l + 2] == 0
        scores.append(jnp.where(maskl, neg, scl))
    m = jnp.maximum(jnp.maximum(scores[0], scores[1]),
                    jnp.maximum(scores[2], scores[3]))
    exps = [jnp.exp(sc - m) for sc in scores]
    denom = exps[0] + exps[1] + exps[2] + exps[3]
    w = [e / denom for e in exps]

    wcat = jnp.concatenate(
        [jnp.broadcast_to(w[1 + l], (S, GMAX)) for l in range(NL)], axis=1)
    out = (es2 * w[0]
           + jnp.dot(oh2 * wcat, mean, preferred_element_type=f32))
    out_ref[0] = out


@jax.jit
def kernel(sequence, group_id, emb, conv_w, proj_w, score_w, score_b):
    f32 = jnp.float32
    idxT = jnp.concatenate(
        [sequence[:, :, None], jnp.transpose(group_id, (0, 2, 1))], axis=2)
    wk = conv_w[:, 0, :].T
    projT = proj_w[:, :, 0].T
    swT = score_w.T
    sb = score_b.reshape(1, 1).astype(f32)

    grid = (B,)
    return pl.pallas_call(
        _gbst_kernel,
        grid=grid,
        in_specs=[
            pl.BlockSpec((1, S, K), lambda b: (b, 0, 0)),
            pl.BlockSpec((V, D), lambda b: (0, 0)),
            pl.BlockSpec((K, D), lambda b: (0, 0)),
            pl.BlockSpec((D, D), lambda b: (0, 0)),
            pl.BlockSpec((D, 1), lambda b: (0, 0)),
            pl.BlockSpec((1, 1), lambda b: (0, 0)),
        ],
        out_specs=pl.BlockSpec((1, S, D), lambda b: (b, 0, 0)),
        out_shape=jax.ShapeDtypeStruct((B, S, D), f32),
        scratch_shapes=[pltpu.VMEM((SP, D), f32)],
        compiler_params=pltpu.CompilerParams(
            dimension_semantics=("parallel",)),
    )(idxT, emb, wk, projT, swT, sb)
